# baseline (device time: 12166 ns/iter reference)
import jax
import jax.numpy as jnp
from jax import lax
from jax.experimental import pallas as pl
from jax.experimental.pallas import tpu as pltpu


def kernel(x):
    m, n = x.shape
    half = n // 2

    def body(
        x_hbm,
        out_hbm,
        xp_vmem,
        xm_vmem,
        send_buf,
        out_local,
        load_p_sem,
        load_m_sem,
        store_sem,
        send_sem,
        recv_sem,
    ):
        my_x = lax.axis_index("x")
        my_y = lax.axis_index("y")
        my_z = lax.axis_index("z")
        partner = 1 - my_x

        barrier_sem = pltpu.get_barrier_semaphore()
        pl.semaphore_signal(
            barrier_sem,
            inc=1,
            device_id=(partner, my_y, my_z),
            device_id_type=pl.DeviceIdType.MESH,
        )

        def exchange(my_cols, partner_cols, my_rows):
            load_p = pltpu.make_async_copy(
                x_hbm.at[:, partner_cols], xp_vmem, load_p_sem
            )
            load_m = pltpu.make_async_copy(
                x_hbm.at[:, my_cols], xm_vmem, load_m_sem
            )
            load_p.start()
            load_m.start()

            load_p.wait()
            send_buf[...] = xp_vmem[...].astype(jnp.bfloat16)
            pl.semaphore_wait(barrier_sem, 1)
            rdma = pltpu.make_async_remote_copy(
                src_ref=send_buf,
                dst_ref=out_hbm.at[my_rows, :],
                send_sem=send_sem,
                recv_sem=recv_sem,
                device_id=(partner, my_y, my_z),
                device_id_type=pl.DeviceIdType.MESH,
            )
            rdma.start()

            load_m.wait()
            out_local[...] = xm_vmem[...].astype(jnp.bfloat16)
            store = pltpu.make_async_copy(
                out_local, out_hbm.at[my_rows, :], store_sem
            )
            store.start()
            store.wait()
            rdma.wait()

        @pl.when(my_x == 0)
        def _():
            exchange(
                my_cols=pl.ds(0, half),
                partner_cols=pl.ds(half, half),
                my_rows=pl.ds(0, m),
            )

        @pl.when(my_x == 1)
        def _():
            exchange(
                my_cols=pl.ds(half, half),
                partner_cols=pl.ds(0, half),
                my_rows=pl.ds(m, m),
            )

    return pl.pallas_call(
        body,
        out_shape=jax.ShapeDtypeStruct((2 * m, half), jnp.bfloat16),
        in_specs=[pl.BlockSpec(memory_space=pltpu.MemorySpace.HBM)],
        out_specs=pl.BlockSpec(memory_space=pltpu.MemorySpace.HBM),
        scratch_shapes=[
            pltpu.VMEM((m, half), x.dtype),
            pltpu.VMEM((m, half), x.dtype),
            pltpu.VMEM((m, half), jnp.bfloat16),
            pltpu.VMEM((m, half), jnp.bfloat16),
            pltpu.SemaphoreType.DMA,
            pltpu.SemaphoreType.DMA,
            pltpu.SemaphoreType.DMA,
            pltpu.SemaphoreType.DMA,
            pltpu.SemaphoreType.DMA,
        ],
        compiler_params=pltpu.CompilerParams(collective_id=0),
    )(x)


# device time: 11921 ns/iter; 1.0206x vs baseline; 1.0206x over previous
import jax
import jax.numpy as jnp
from jax import lax
from jax.experimental import pallas as pl
from jax.experimental.pallas import tpu as pltpu


def kernel(x):
    m, n = x.shape
    half = n // 2

    def body(x_ref, out_ref, send_sem, recv_sem):
        my_x = lax.axis_index("x")
        my_y = lax.axis_index("y")
        my_z = lax.axis_index("z")
        partner = 1 - my_x

        barrier_sem = pltpu.get_barrier_semaphore()
        pl.semaphore_signal(
            barrier_sem,
            inc=1,
            device_id=(partner, my_y, my_z),
            device_id_type=pl.DeviceIdType.MESH,
        )
        pl.semaphore_wait(barrier_sem, 1)

        def exchange(my_cols, partner_cols, my_rows):
            rdma = pltpu.make_async_remote_copy(
                src_ref=x_ref.at[:, partner_cols],
                dst_ref=out_ref.at[my_rows, :],
                send_sem=send_sem,
                recv_sem=recv_sem,
                device_id=(partner, my_y, my_z),
                device_id_type=pl.DeviceIdType.MESH,
            )
            rdma.start()
            out_ref[my_rows, :] = x_ref[:, my_cols]
            rdma.wait()

        @pl.when(my_x == 0)
        def _():
            exchange(
                my_cols=pl.ds(0, half),
                partner_cols=pl.ds(half, half),
                my_rows=pl.ds(0, m),
            )

        @pl.when(my_x == 1)
        def _():
            exchange(
                my_cols=pl.ds(half, half),
                partner_cols=pl.ds(0, half),
                my_rows=pl.ds(m, m),
            )

    return pl.pallas_call(
        body,
        out_shape=jax.ShapeDtypeStruct((2 * m, half), jnp.bfloat16),
        in_specs=[pl.BlockSpec(memory_space=pltpu.MemorySpace.VMEM)],
        out_specs=pl.BlockSpec(memory_space=pltpu.MemorySpace.VMEM),
        scratch_shapes=[
            pltpu.SemaphoreType.DMA,
            pltpu.SemaphoreType.DMA,
        ],
        compiler_params=pltpu.CompilerParams(collective_id=0),
    )(x.astype(jnp.bfloat16))


# device time: 11797 ns/iter; 1.0313x vs baseline; 1.0105x over previous
import jax
import jax.numpy as jnp
from jax import lax
from jax.experimental import pallas as pl
from jax.experimental.pallas import tpu as pltpu

N_CHUNKS = 4


def kernel(x):
    m, n = x.shape
    half = n // 2
    chunk = half // N_CHUNKS

    def body(x_ref, out_ref, send_buf, send_sems, recv_sems):
        my_x = lax.axis_index("x")
        my_y = lax.axis_index("y")
        my_z = lax.axis_index("z")
        partner = 1 - my_x

        barrier_sem = pltpu.get_barrier_semaphore()
        pl.semaphore_signal(
            barrier_sem,
            inc=1,
            device_id=(partner, my_y, my_z),
            device_id_type=pl.DeviceIdType.MESH,
        )

        def exchange(my_base, partner_base, my_row0):
            rdmas = []
            for c in range(N_CHUNKS):
                lo = c * chunk
                send_buf[:, pl.ds(lo, chunk)] = x_ref[
                    :, pl.ds(partner_base + lo, chunk)
                ].astype(jnp.bfloat16)
                if c == 0:
                    pl.semaphore_wait(barrier_sem, 1)
                rdma = pltpu.make_async_remote_copy(
                    src_ref=send_buf.at[:, pl.ds(lo, chunk)],
                    dst_ref=out_ref.at[pl.ds(my_row0, m), pl.ds(lo, chunk)],
                    send_sem=send_sems.at[c],
                    recv_sem=recv_sems.at[c],
                    device_id=(partner, my_y, my_z),
                    device_id_type=pl.DeviceIdType.MESH,
                )
                rdma.start()
                rdmas.append(rdma)
            out_ref[pl.ds(my_row0, m), :] = x_ref[
                :, pl.ds(my_base, half)
            ].astype(jnp.bfloat16)
            for rdma in rdmas:
                rdma.wait()

        @pl.when(my_x == 0)
        def _():
            exchange(my_base=0, partner_base=half, my_row0=0)

        @pl.when(my_x == 1)
        def _():
            exchange(my_base=half, partner_base=0, my_row0=m)

    return pl.pallas_call(
        body,
        out_shape=jax.ShapeDtypeStruct((2 * m, half), jnp.bfloat16),
        in_specs=[pl.BlockSpec(memory_space=pltpu.MemorySpace.VMEM)],
        out_specs=pl.BlockSpec(memory_space=pltpu.MemorySpace.VMEM),
        scratch_shapes=[
            pltpu.VMEM((m, half), jnp.bfloat16),
            pltpu.SemaphoreType.DMA((N_CHUNKS,)),
            pltpu.SemaphoreType.DMA((N_CHUNKS,)),
        ],
        compiler_params=pltpu.CompilerParams(collective_id=0),
    )(x)
